# baseline (speedup 1.0000x reference)
import functools

import numpy as np

import jax
import jax.numpy as jnp
from jax.experimental import pallas as pl
from jax.experimental.pallas import tpu as pltpu

D_MODEL = 1024
N_HEADS = 16
D_K = 64
HALF = 32
IND_HEADS = 4
IND_DIM = 64
SEQ = 2048
TILE = 128
N_TILES = SEQ // TILE
TOPK_BASE = 512.0
GMIN = 0.5
GMAX = 1.5
ROPE_B = 10000.0

_NEG = -1e30


def _i32(v):
    return int(np.uint32(v & 0xFFFFFFFF).view(np.int32))


TP = 256
NP_TILES = SEQ // TP


def _proj_kernel(x_ref, wqkv_ref, wind_ref, cos_ref, sin_ref,
                 q_ref, kt_ref, v_ref, iq_ref, ikt_ref, iw_ref):
    x = x_ref[...]
    xw = jnp.dot(x, wqkv_ref[...], preferred_element_type=jnp.float32)
    xi = jnp.dot(x, wind_ref[...], preferred_element_type=jnp.float32)
    cos_i = jnp.concatenate([cos_ref[...]] * N_HEADS, axis=1)
    sin_i = jnp.concatenate([sin_ref[...]] * N_HEADS, axis=1)
    lane = jax.lax.broadcasted_iota(jnp.int32, (TP, D_MODEL), 1)
    even = (lane & 1) == 0

    def rope(zz):
        sw = jnp.where(even, jnp.roll(zz, -1, axis=1), jnp.roll(zz, 1, axis=1))
        return zz * cos_i + sw * sin_i

    q_ref[...] = (rope(xw[:, 0:1024]) * 0.125).astype(jnp.bfloat16)
    kt_ref[...] = rope(xw[:, 1024:2048]).astype(jnp.bfloat16).T
    v_ref[...] = xw[:, 2048:3072].astype(jnp.bfloat16)
    iq_ref[...] = xi[:, 0:256]
    ikt_ref[...] = xi[:, 256:320].T
    iw_ref[...] = xi[:, 320:384]


def _run_proj(x2, wqkv, wind, cos_t, sin_t):
    return pl.pallas_call(
        _proj_kernel,
        grid=(NP_TILES,),
        in_specs=[
            pl.BlockSpec((TP, D_MODEL), lambda i: (i, 0)),
            pl.BlockSpec((D_MODEL, 3072), lambda i: (0, 0)),
            pl.BlockSpec((D_MODEL, 448), lambda i: (0, 0)),
            pl.BlockSpec((TP, D_K), lambda i: (i, 0)),
            pl.BlockSpec((TP, D_K), lambda i: (i, 0)),
        ],
        out_specs=[
            pl.BlockSpec((TP, D_MODEL), lambda i: (i, 0)),
            pl.BlockSpec((D_MODEL, TP), lambda i: (0, i)),
            pl.BlockSpec((TP, D_MODEL), lambda i: (i, 0)),
            pl.BlockSpec((TP, 256), lambda i: (i, 0)),
            pl.BlockSpec((IND_DIM, TP), lambda i: (0, i)),
            pl.BlockSpec((TP, 64), lambda i: (i, 0)),
        ],
        out_shape=[
            jax.ShapeDtypeStruct((SEQ, D_MODEL), jnp.bfloat16),
            jax.ShapeDtypeStruct((D_MODEL, SEQ), jnp.bfloat16),
            jax.ShapeDtypeStruct((SEQ, D_MODEL), jnp.bfloat16),
            jax.ShapeDtypeStruct((SEQ, 256), jnp.float32),
            jax.ShapeDtypeStruct((IND_DIM, SEQ), jnp.float32),
            jax.ShapeDtypeStruct((SEQ, 64), jnp.float32),
        ],
    )(x2, wqkv, wind, cos_t, sin_t)


def _scores_kernel(x_ref, iq_ref, ikt_ref, iw_ref, w1_ref, b1_ref, w2_ref,
                   b2_ref, sc_ref, ent_ref, g_ref):
    ti = pl.program_id(0)
    ikt = ikt_ref[...]
    acc = jnp.zeros((TP, SEQ), jnp.float32)
    for h in range(IND_HEADS):
        iqh = iq_ref[:, h * IND_DIM:(h + 1) * IND_DIM]
        dots = jnp.dot(iqh, ikt, preferred_element_type=jnp.float32)
        acc = acc + jnp.maximum(dots, 0.0) * iw_ref[:, h:h + 1]
    rows = ti * TP + jax.lax.broadcasted_iota(jnp.int32, (TP, SEQ), 0)
    cols = jax.lax.broadcasted_iota(jnp.int32, (TP, SEQ), 1)
    masked = jnp.where(cols > rows, -jnp.inf, acc)
    sc_ref[...] = masked
    m = jnp.max(masked, axis=1, keepdims=True)
    e = jnp.exp(masked - m)
    z = jnp.sum(e, axis=1, keepdims=True)
    p = e / z
    ent_rows = -jnp.sum(p * jnp.log(p + 1e-9), axis=1)

    @pl.when(ti == 0)
    def _():
        ent_ref[0, 0] = 0.0

    ent_ref[0, 0] += jnp.sum(ent_rows)

    @pl.when(ti == NP_TILES - 1)
    def _():
        pooled = jnp.mean(x_ref[...], axis=0, keepdims=True)
        h1 = jnp.maximum(
            jnp.dot(pooled, w1_ref[...], preferred_element_type=jnp.float32)
            + b1_ref[...], 0.0)
        o = (jnp.dot(h1, w2_ref[...], preferred_element_type=jnp.float32)
             + b2_ref[...])
        g_ref[0, 0] = jax.nn.sigmoid(o)[0, 0]


def _run_scores(x2, iq, ikt, iw, wg1, bg1, wg2, bg2):
    gh = wg1.shape[1]
    return pl.pallas_call(
        _scores_kernel,
        grid=(NP_TILES,),
        in_specs=[
            pl.BlockSpec((SEQ, D_MODEL), lambda i: (0, 0)),
            pl.BlockSpec((TP, 256), lambda i: (i, 0)),
            pl.BlockSpec((IND_DIM, SEQ), lambda i: (0, 0)),
            pl.BlockSpec((TP, 64), lambda i: (i, 0)),
            pl.BlockSpec((D_MODEL, gh), lambda i: (0, 0)),
            pl.BlockSpec((1, gh), lambda i: (0, 0)),
            pl.BlockSpec((gh, 1), lambda i: (0, 0)),
            pl.BlockSpec((1, 1), lambda i: (0, 0)),
        ],
        out_specs=[
            pl.BlockSpec((TP, SEQ), lambda i: (i, 0)),
            pl.BlockSpec(memory_space=pltpu.SMEM),
            pl.BlockSpec(memory_space=pltpu.SMEM),
        ],
        out_shape=[
            jax.ShapeDtypeStruct((SEQ, SEQ), jnp.float32),
            jax.ShapeDtypeStruct((1, 1), jnp.float32),
            jax.ShapeDtypeStruct((1, 1), jnp.float32),
        ],
    )(x2, iq, ikt, iw, wg1, bg1.reshape(1, gh), wg2, bg2.reshape(1, 1))


def _select_kernel(k_ref, sc_ref, mask_ref):
    i = pl.program_id(0)
    kval = k_ref[0, 0]
    sign_bit = jnp.int32(_i32(0x80000000))
    idx_full = jax.lax.broadcasted_iota(jnp.int32, (TILE, SEQ), 1)

    @pl.when(kval >= (i + 1) * TILE)
    def _trivial():
        mask_ref[...] = (idx_full < kval).astype(jnp.int8)

    def radix(width):
        sv = sc_ref[:, :width] + 0.0
        b = jax.lax.bitcast_convert_type(sv, jnp.int32)
        u = jnp.where(b < 0, jnp.bitwise_not(b),
                      jnp.bitwise_or(b, sign_bit))
        idx = jax.lax.broadcasted_iota(jnp.int32, (TILE, width), 1)

        def count(hit):
            return jnp.sum(hit.astype(jnp.int32), axis=1, keepdims=True)

        k_rem = jnp.full((TILE, 1), kval, jnp.int32)
        cnt1 = count(u < 0)
        take = cnt1 >= k_rem
        p_hi = jnp.where(take, sign_bit, 0)
        k_rem = jnp.where(take, k_rem, k_rem - cnt1)
        for j in range(30, -1, -1):
            bit = jnp.int32(1 << j)
            cnt1 = count(((u ^ p_hi) >> j) == 1)
            take = cnt1 >= k_rem
            p_hi = jnp.where(take, p_hi | bit, p_hi)
            k_rem = jnp.where(take, k_rem, k_rem - cnt1)

        t_s = p_hi ^ sign_bit
        quota = k_rem
        eq = u == p_hi
        q_hi = jnp.zeros((TILE, 1), jnp.int32)
        for j in range(10, -1, -1):
            bit = jnp.int32(1 << j)
            c0 = count(eq & (((idx ^ q_hi) >> j) == 0))
            take0 = quota <= c0
            q_hi = jnp.where(take0, q_hi, q_hi | bit)
            quota = jnp.where(take0, quota, quota - c0)

        allowed = (((u ^ sign_bit) > t_s) | (eq & (idx <= q_hi)))
        mask_ref[:, :width] = allowed.astype(jnp.int8)
        if width < SEQ:
            mask_ref[:, width:] = jnp.zeros((TILE, SEQ - width), jnp.int8)

    nontriv = kval < (i + 1) * TILE
    for width, lo_t, hi_t in ((512, 0, 4), (1024, 4, 8), (2048, 8, 16)):
        @pl.when(nontriv & (i >= lo_t) & (i < hi_t))
        def _(width=width):
            radix(width)


def _run_select(scores, kscal):
    return pl.pallas_call(
        _select_kernel,
        grid=(N_TILES,),
        in_specs=[
            pl.BlockSpec(memory_space=pltpu.SMEM),
            pl.BlockSpec((TILE, SEQ), lambda i: (i, 0)),
        ],
        out_specs=pl.BlockSpec((TILE, SEQ), lambda i: (i, 0)),
        out_shape=jax.ShapeDtypeStruct((SEQ, SEQ), jnp.int8),
    )(kscal, scores)


_CHUNK = 512
_N_CHUNKS = SEQ // _CHUNK


def _attn_kernel(q_ref, kt_ref, v_ref, mask_ref, wo_ref, o_ref, acc_ref):
    ok = mask_ref[...] != 0
    for h in range(N_HEADS):
        qh = q_ref[:, h * D_K:(h + 1) * D_K]
        kth = kt_ref[h * D_K:(h + 1) * D_K, :]
        logits = jnp.dot(qh, kth,
                         preferred_element_type=jnp.float32).astype(jnp.bfloat16)
        logits = jnp.where(ok, logits, jnp.bfloat16(_NEG))
        m = jnp.max(logits, axis=1, keepdims=True)
        e = jnp.exp(logits - m)
        z = jnp.sum(e.astype(jnp.float32), axis=1, keepdims=True)
        vh = v_ref[:, h * D_K:(h + 1) * D_K]
        acc_ref[:, h * D_K:(h + 1) * D_K] = jnp.dot(
            e, vh, preferred_element_type=jnp.float32) / z
    o_ref[...] = jnp.dot(acc_ref[...], wo_ref[...],
                         preferred_element_type=jnp.float32)


def _run_attn(q, kt, v, mask, wo, kscal):
    del kscal
    return pl.pallas_call(
        _attn_kernel,
        grid=(NP_TILES,),
        in_specs=[
            pl.BlockSpec((TP, D_MODEL), lambda i: (i, 0)),
            pl.BlockSpec((D_MODEL, SEQ), lambda i: (0, 0)),
            pl.BlockSpec((SEQ, D_MODEL), lambda i: (0, 0)),
            pl.BlockSpec((TP, SEQ), lambda i: (i, 0)),
            pl.BlockSpec((D_MODEL, D_MODEL), lambda i: (0, 0)),
        ],
        out_specs=pl.BlockSpec((TP, D_MODEL), lambda i: (i, 0)),
        out_shape=jax.ShapeDtypeStruct((SEQ, D_MODEL), jnp.float32),
        scratch_shapes=[pltpu.VMEM((TP, D_MODEL), jnp.float32)],
    )(q, kt, v, mask, wo)


def _two_prod_(a, b):
    p = a * b
    c = jnp.float32(4097.0)
    a_c = a * c
    a_hi = a_c - (a_c - a)
    a_lo = a - a_hi
    b_c = b * c
    b_hi = b_c - (b_c - b)
    b_lo = b - b_hi
    err = ((a_hi * b_hi - p) + a_hi * b_lo + a_lo * b_hi) + a_lo * b_lo
    return p, err


def _exact_floor_prod_(a, b):
    p, err = _two_prod_(a, b)
    base = jnp.floor(p)
    r = p - base
    t = r + err
    base = base + jnp.where(t >= 1.0, 1.0, 0.0) - jnp.where(t < 0.0, 1.0, 0.0)
    return base


def kernel(x, Wqkv, Wo, Wiq, Wik, Wiw, Wg1, bg1, Wg2, bg2):
    b, s, d = x.shape
    x2 = x[0]

    wiw_pad = jnp.pad(Wiw, ((0, 0), (0, 128 - IND_HEADS)))
    wind = jnp.concatenate([Wiq, Wik, wiw_pad], axis=1)

    theta = 1.0 / (ROPE_B ** (jnp.arange(HALF, dtype=jnp.float32) * 2.0 / D_K))
    ang = jnp.arange(s, dtype=jnp.float32)[:, None] * theta[None, :]
    cos_t = jnp.repeat(jnp.cos(ang), 2, axis=1)
    sgn = jnp.tile(jnp.array([-1.0, 1.0], jnp.float32), (HALF,))
    sin_t = jnp.repeat(jnp.sin(ang), 2, axis=1) * sgn[None, :]

    q, kt, v, iq, ikt, iw = _run_proj(x2, Wqkv, wind, cos_t, sin_t)
    scores, ent_sum, g = _run_scores(x2, iq, ikt, iw, Wg1, bg1, Wg2, bg2)

    ent_mean = ent_sum[0, 0] / jnp.float32(s)
    entropy_norm = ent_mean / jnp.log(float(s))
    entropy_factor = jnp.clip(GMIN + entropy_norm, GMIN, GMAX)
    gate_factor = GMIN + (GMAX - GMIN) * g[0, 0]
    scaled_gate = jnp.float32(TOPK_BASE) * gate_factor
    adaptive_k = _exact_floor_prod_(scaled_gate, entropy_factor).astype(jnp.int32)
    adaptive_k = jnp.clip(adaptive_k, 1, s)

    kscal = adaptive_k.reshape(1, 1)
    mask = _run_select(scores, kscal)
    y = _run_attn(q, kt, v, mask, Wo, kscal)
    return y.reshape(b, s, d)


# speedup vs baseline: 1.0471x; 1.0471x over previous
import functools

import numpy as np

import jax
import jax.numpy as jnp
from jax.experimental import pallas as pl
from jax.experimental.pallas import tpu as pltpu

D_MODEL = 1024
N_HEADS = 16
D_K = 64
HALF = 32
IND_HEADS = 4
IND_DIM = 64
SEQ = 2048
TILE = 128
N_TILES = SEQ // TILE
TOPK_BASE = 512.0
GMIN = 0.5
GMAX = 1.5
ROPE_B = 10000.0

_NEG = -1e30


def _i32(v):
    return int(np.uint32(v & 0xFFFFFFFF).view(np.int32))


TP = 256
NP_TILES = SEQ // TP


def _proj_kernel(x_ref, wqkv_ref, wind_ref, cos_ref, sin_ref,
                 q_ref, kt_ref, v_ref, iq_ref, ikt_ref, iw_ref):
    x = x_ref[...]
    xw = jnp.dot(x, wqkv_ref[...], preferred_element_type=jnp.float32)
    xi = jnp.dot(x, wind_ref[...], preferred_element_type=jnp.float32)
    cos_i = jnp.concatenate([cos_ref[...]] * N_HEADS, axis=1)
    sin_i = jnp.concatenate([sin_ref[...]] * N_HEADS, axis=1)
    lane = jax.lax.broadcasted_iota(jnp.int32, (TP, D_MODEL), 1)
    even = (lane & 1) == 0

    def rope(zz):
        sw = jnp.where(even, jnp.roll(zz, -1, axis=1), jnp.roll(zz, 1, axis=1))
        return zz * cos_i + sw * sin_i

    q_ref[...] = (rope(xw[:, 0:1024]) * 0.125).astype(jnp.bfloat16)
    kt_ref[...] = rope(xw[:, 1024:2048]).astype(jnp.bfloat16).T
    v_ref[...] = xw[:, 2048:3072].astype(jnp.bfloat16)
    iq_ref[...] = xi[:, 0:256]
    ikt_ref[...] = xi[:, 256:320].T
    iw_ref[...] = xi[:, 320:384]


def _run_proj(x2, wqkv, wind, cos_t, sin_t):
    return pl.pallas_call(
        _proj_kernel,
        grid=(NP_TILES,),
        in_specs=[
            pl.BlockSpec((TP, D_MODEL), lambda i: (i, 0)),
            pl.BlockSpec((D_MODEL, 3072), lambda i: (0, 0)),
            pl.BlockSpec((D_MODEL, 448), lambda i: (0, 0)),
            pl.BlockSpec((TP, D_K), lambda i: (i, 0)),
            pl.BlockSpec((TP, D_K), lambda i: (i, 0)),
        ],
        out_specs=[
            pl.BlockSpec((TP, D_MODEL), lambda i: (i, 0)),
            pl.BlockSpec((D_MODEL, TP), lambda i: (0, i)),
            pl.BlockSpec((TP, D_MODEL), lambda i: (i, 0)),
            pl.BlockSpec((TP, 256), lambda i: (i, 0)),
            pl.BlockSpec((IND_DIM, TP), lambda i: (0, i)),
            pl.BlockSpec((TP, 64), lambda i: (i, 0)),
        ],
        out_shape=[
            jax.ShapeDtypeStruct((SEQ, D_MODEL), jnp.bfloat16),
            jax.ShapeDtypeStruct((D_MODEL, SEQ), jnp.bfloat16),
            jax.ShapeDtypeStruct((SEQ, D_MODEL), jnp.bfloat16),
            jax.ShapeDtypeStruct((SEQ, 256), jnp.float32),
            jax.ShapeDtypeStruct((IND_DIM, SEQ), jnp.float32),
            jax.ShapeDtypeStruct((SEQ, 64), jnp.float32),
        ],
    )(x2, wqkv, wind, cos_t, sin_t)


def _scores_kernel(x_ref, iq_ref, ikt_ref, iw_ref, w1_ref, b1_ref, w2_ref,
                   b2_ref, sc_ref, ent_ref, g_ref):
    ti = pl.program_id(0)
    ikt = ikt_ref[...]
    acc = jnp.zeros((TP, SEQ), jnp.float32)
    for h in range(IND_HEADS):
        iqh = iq_ref[:, h * IND_DIM:(h + 1) * IND_DIM]
        dots = jnp.dot(iqh, ikt, preferred_element_type=jnp.float32)
        acc = acc + jnp.maximum(dots, 0.0) * iw_ref[:, h:h + 1]
    rows = ti * TP + jax.lax.broadcasted_iota(jnp.int32, (TP, SEQ), 0)
    cols = jax.lax.broadcasted_iota(jnp.int32, (TP, SEQ), 1)
    masked = jnp.where(cols > rows, -jnp.inf, acc)
    sc_ref[...] = masked
    m = jnp.max(masked, axis=1, keepdims=True)
    e = jnp.exp(masked - m)
    z = jnp.sum(e, axis=1, keepdims=True)
    p = e / z
    ent_rows = -jnp.sum(p * jnp.log(p + 1e-9), axis=1)

    @pl.when(ti == 0)
    def _():
        ent_ref[0, 0] = 0.0

    ent_ref[0, 0] += jnp.sum(ent_rows)

    @pl.when(ti == NP_TILES - 1)
    def _():
        pooled = jnp.mean(x_ref[...], axis=0, keepdims=True)
        h1 = jnp.maximum(
            jnp.dot(pooled, w1_ref[...], preferred_element_type=jnp.float32)
            + b1_ref[...], 0.0)
        o = (jnp.dot(h1, w2_ref[...], preferred_element_type=jnp.float32)
             + b2_ref[...])
        g_ref[0, 0] = jax.nn.sigmoid(o)[0, 0]


def _run_scores(x2, iq, ikt, iw, wg1, bg1, wg2, bg2):
    gh = wg1.shape[1]
    return pl.pallas_call(
        _scores_kernel,
        grid=(NP_TILES,),
        in_specs=[
            pl.BlockSpec((SEQ, D_MODEL), lambda i: (0, 0)),
            pl.BlockSpec((TP, 256), lambda i: (i, 0)),
            pl.BlockSpec((IND_DIM, SEQ), lambda i: (0, 0)),
            pl.BlockSpec((TP, 64), lambda i: (i, 0)),
            pl.BlockSpec((D_MODEL, gh), lambda i: (0, 0)),
            pl.BlockSpec((1, gh), lambda i: (0, 0)),
            pl.BlockSpec((gh, 1), lambda i: (0, 0)),
            pl.BlockSpec((1, 1), lambda i: (0, 0)),
        ],
        out_specs=[
            pl.BlockSpec((TP, SEQ), lambda i: (i, 0)),
            pl.BlockSpec(memory_space=pltpu.SMEM),
            pl.BlockSpec(memory_space=pltpu.SMEM),
        ],
        out_shape=[
            jax.ShapeDtypeStruct((SEQ, SEQ), jnp.float32),
            jax.ShapeDtypeStruct((1, 1), jnp.float32),
            jax.ShapeDtypeStruct((1, 1), jnp.float32),
        ],
    )(x2, iq, ikt, iw, wg1, bg1.reshape(1, gh), wg2, bg2.reshape(1, 1))


def _make_select_kernel(width, base):
    def _select_kernel(k_ref, sc_ref, mask_ref):
        i = base + pl.program_id(0)
        kval = k_ref[0, 0]
        sign_bit = jnp.int32(_i32(0x80000000))
        idx_full = jax.lax.broadcasted_iota(jnp.int32, (TILE, SEQ), 1)

        @pl.when(kval >= (i + 1) * TILE)
        def _trivial():
            mask_ref[...] = (idx_full < kval).astype(jnp.int8)

        @pl.when(kval < (i + 1) * TILE)
        def _radix():
            sv = sc_ref[...] + 0.0
            b = jax.lax.bitcast_convert_type(sv, jnp.int32)
            u = jnp.where(b < 0, jnp.bitwise_not(b),
                          jnp.bitwise_or(b, sign_bit))
            idx = jax.lax.broadcasted_iota(jnp.int32, (TILE, width), 1)

            def count(hit):
                return jnp.sum(hit.astype(jnp.int32), axis=1, keepdims=True)

            k_rem = jnp.full((TILE, 1), kval, jnp.int32)
            cnt1 = count(u < 0)
            take = cnt1 >= k_rem
            p_hi = jnp.where(take, sign_bit, 0)
            k_rem = jnp.where(take, k_rem, k_rem - cnt1)
            for j in range(30, -1, -1):
                bit = jnp.int32(1 << j)
                cnt1 = count(((u ^ p_hi) >> j) == 1)
                take = cnt1 >= k_rem
                p_hi = jnp.where(take, p_hi | bit, p_hi)
                k_rem = jnp.where(take, k_rem, k_rem - cnt1)

            t_s = p_hi ^ sign_bit
            quota = k_rem
            eq = u == p_hi
            q_hi = jnp.zeros((TILE, 1), jnp.int32)
            for j in range(10, -1, -1):
                bit = jnp.int32(1 << j)
                c0 = count(eq & (((idx ^ q_hi) >> j) == 0))
                take0 = quota <= c0
                q_hi = jnp.where(take0, q_hi, q_hi | bit)
                quota = jnp.where(take0, quota, quota - c0)

            allowed = (((u ^ sign_bit) > t_s) | (eq & (idx <= q_hi)))
            mask_ref[:, :width] = allowed.astype(jnp.int8)
            if width < SEQ:
                mask_ref[:, width:] = jnp.zeros((TILE, SEQ - width),
                                                jnp.int8)

    return _select_kernel


def _run_select(scores, kscal):
    pieces = []
    for width, base, ntiles in ((512, 0, 4), (1024, 4, 4), (2048, 8, 8)):
        pieces.append(pl.pallas_call(
            _make_select_kernel(width, base),
            grid=(ntiles,),
            in_specs=[
                pl.BlockSpec(memory_space=pltpu.SMEM),
                pl.BlockSpec((TILE, width),
                             lambda i, base=base: (base + i, 0)),
            ],
            out_specs=pl.BlockSpec((TILE, SEQ), lambda i: (i, 0)),
            out_shape=jax.ShapeDtypeStruct((ntiles * TILE, SEQ), jnp.int8),
        )(kscal, scores))
    return jnp.concatenate(pieces, axis=0)


_CHUNK = 512
_N_CHUNKS = SEQ // _CHUNK


def _attn_kernel(q_ref, kt_ref, v_ref, mask_ref, wo_ref, o_ref, acc_ref):
    ok = mask_ref[...] != 0
    for h in range(N_HEADS):
        qh = q_ref[:, h * D_K:(h + 1) * D_K]
        kth = kt_ref[h * D_K:(h + 1) * D_K, :]
        logits = jnp.dot(qh, kth,
                         preferred_element_type=jnp.float32).astype(jnp.bfloat16)
        logits = jnp.where(ok, logits, jnp.bfloat16(_NEG))
        m = jnp.max(logits, axis=1, keepdims=True)
        e = jnp.exp(logits - m)
        z = jnp.sum(e.astype(jnp.float32), axis=1, keepdims=True)
        vh = v_ref[:, h * D_K:(h + 1) * D_K]
        acc_ref[:, h * D_K:(h + 1) * D_K] = jnp.dot(
            e, vh, preferred_element_type=jnp.float32) / z
    o_ref[...] = jnp.dot(acc_ref[...], wo_ref[...],
                         preferred_element_type=jnp.float32)


def _run_attn(q, kt, v, mask, wo, kscal):
    del kscal
    return pl.pallas_call(
        _attn_kernel,
        grid=(NP_TILES,),
        in_specs=[
            pl.BlockSpec((TP, D_MODEL), lambda i: (i, 0)),
            pl.BlockSpec((D_MODEL, SEQ), lambda i: (0, 0)),
            pl.BlockSpec((SEQ, D_MODEL), lambda i: (0, 0)),
            pl.BlockSpec((TP, SEQ), lambda i: (i, 0)),
            pl.BlockSpec((D_MODEL, D_MODEL), lambda i: (0, 0)),
        ],
        out_specs=pl.BlockSpec((TP, D_MODEL), lambda i: (i, 0)),
        out_shape=jax.ShapeDtypeStruct((SEQ, D_MODEL), jnp.float32),
        scratch_shapes=[pltpu.VMEM((TP, D_MODEL), jnp.float32)],
    )(q, kt, v, mask, wo)


def _two_prod_(a, b):
    p = a * b
    c = jnp.float32(4097.0)
    a_c = a * c
    a_hi = a_c - (a_c - a)
    a_lo = a - a_hi
    b_c = b * c
    b_hi = b_c - (b_c - b)
    b_lo = b - b_hi
    err = ((a_hi * b_hi - p) + a_hi * b_lo + a_lo * b_hi) + a_lo * b_lo
    return p, err


def _exact_floor_prod_(a, b):
    p, err = _two_prod_(a, b)
    base = jnp.floor(p)
    r = p - base
    t = r + err
    base = base + jnp.where(t >= 1.0, 1.0, 0.0) - jnp.where(t < 0.0, 1.0, 0.0)
    return base


def kernel(x, Wqkv, Wo, Wiq, Wik, Wiw, Wg1, bg1, Wg2, bg2):
    b, s, d = x.shape
    x2 = x[0]

    wiw_pad = jnp.pad(Wiw, ((0, 0), (0, 128 - IND_HEADS)))
    wind = jnp.concatenate([Wiq, Wik, wiw_pad], axis=1)

    theta = 1.0 / (ROPE_B ** (jnp.arange(HALF, dtype=jnp.float32) * 2.0 / D_K))
    ang = jnp.arange(s, dtype=jnp.float32)[:, None] * theta[None, :]
    cos_t = jnp.repeat(jnp.cos(ang), 2, axis=1)
    sgn = jnp.tile(jnp.array([-1.0, 1.0], jnp.float32), (HALF,))
    sin_t = jnp.repeat(jnp.sin(ang), 2, axis=1) * sgn[None, :]

    q, kt, v, iq, ikt, iw = _run_proj(x2, Wqkv, wind, cos_t, sin_t)
    scores, ent_sum, g = _run_scores(x2, iq, ikt, iw, Wg1, bg1, Wg2, bg2)

    ent_mean = ent_sum[0, 0] / jnp.float32(s)
    entropy_norm = ent_mean / jnp.log(float(s))
    entropy_factor = jnp.clip(GMIN + entropy_norm, GMIN, GMAX)
    gate_factor = GMIN + (GMAX - GMIN) * g[0, 0]
    scaled_gate = jnp.float32(TOPK_BASE) * gate_factor
    adaptive_k = _exact_floor_prod_(scaled_gate, entropy_factor).astype(jnp.int32)
    adaptive_k = jnp.clip(adaptive_k, 1, s)

    kscal = adaptive_k.reshape(1, 1)
    mask = _run_select(scores, kscal)
    y = _run_attn(q, kt, v, mask, Wo, kscal)
    return y.reshape(b, s, d)
